# baseline (speedup 1.0000x reference)
import functools

import jax
import jax.numpy as jnp
from jax import lax
from jax.experimental import pallas as pl
from jax.experimental.pallas import tpu as pltpu
from jax.experimental.pallas import tpu_sc as plsc

B, L, D = 4096, 200, 128
N = B * L
NUM_PAIR = 144

_INFO = plsc.get_sparse_core_info()
NC = _INFO.num_cores
NS = _INFO.num_subcores
NW = NC * NS
ROWS_PER_W = N // NW
CHUNK = 256
NCHUNK = ROWS_PER_W // CHUNK
GROUPS = CHUNK // 16
NBUF = 2


def _make_sc_call():
  mesh = plsc.VectorSubcoreMesh(core_axis_name="c", subcore_axis_name="s")

  @functools.partial(
      pl.kernel,
      out_type=jax.ShapeDtypeStruct((N * D,), jnp.float32),
      mesh=mesh,
      compiler_params=pltpu.CompilerParams(needs_layout_passes=False),
      scratch_types=[
          pltpu.VMEM((NUM_PAIR * D,), jnp.float32),
          [pltpu.VMEM((CHUNK,), jnp.int32) for _ in range(NBUF)],
          [pltpu.VMEM((CHUNK,), jnp.int32) for _ in range(NBUF)],
          [pltpu.VMEM((CHUNK * D,), jnp.float32) for _ in range(NBUF)],
          [pltpu.SemaphoreType.DMA for _ in range(NBUF)],
          [pltpu.SemaphoreType.DMA for _ in range(NBUF)],
          pltpu.SemaphoreType.DMA,
      ],
  )
  def sc_kernel(t_hbm, p_hbm, q_hbm, out_hbm, t_v, p_v, q_v, o_v,
                sem_i, sem_o, sem_t):
    wid = lax.axis_index("s") * NC + lax.axis_index("c")
    row0 = wid * ROWS_PER_W
    pltpu.async_copy(t_hbm, t_v, sem_t).wait()
    lane = lax.iota(jnp.int32, 16)
    soff0 = lane * D
    rot = [(lane + j) & 15 for j in range(16)]

    def idx_start(k, b):
      base = row0 + k * CHUNK
      pltpu.async_copy(p_hbm.at[pl.ds(base, CHUNK)], p_v[b], sem_i[b])
      pltpu.async_copy(q_hbm.at[pl.ds(base, CHUNK)], q_v[b], sem_i[b])

    def idx_wait(k, b):
      base = row0 + k * CHUNK
      pltpu.make_async_copy(p_hbm.at[pl.ds(base, CHUNK)], p_v[b], sem_i[b]).wait()
      pltpu.make_async_copy(q_hbm.at[pl.ds(base, CHUNK)], q_v[b], sem_i[b]).wait()

    def out_start(k, b):
      base = row0 + k * CHUNK
      pltpu.async_copy(o_v[b], out_hbm.at[pl.ds(base * D, CHUNK * D)], sem_o[b])

    def out_wait(k, b):
      base = row0 + k * CHUNK
      pltpu.make_async_copy(
          o_v[b], out_hbm.at[pl.ds(base * D, CHUNK * D)], sem_o[b]).wait()

    def compute(b):
      @plsc.parallel_loop(0, GROUPS, 1, unroll=2)
      def group_body(g):
        pv = p_v[b][pl.ds(g * 16, 16)] * D
        qv = q_v[b][pl.ds(g * 16, 16)] * D
        ov = soff0 + g * (16 * D)
        for cb in range(8):
          c0 = cb * 16
          pvc = pv + c0
          qvc = qv + c0
          ovc = ov + c0
          for j in range(16):
            r = (plsc.load_gather(t_v, [pvc + rot[j]]) +
                 plsc.load_gather(t_v, [qvc + rot[j]]))
            plsc.store_scatter(o_v[b], [ovc + rot[j]], r)

    for b in range(NBUF):
      idx_start(b, b)

    def chunk_pair(kk, _):
      for b in range(NBUF):
        k = kk * NBUF + b
        idx_wait(k, b)

        @pl.when(k >= NBUF)
        def _():
          out_wait(k - NBUF, b)

        compute(b)
        out_start(k, b)

        @pl.when(k + NBUF < NCHUNK)
        def _():
          idx_start(k + NBUF, b)
      return 0

    lax.fori_loop(0, NCHUNK // NBUF, chunk_pair, 0)
    for b in range(NBUF):
      out_wait(NCHUNK - NBUF + b, b)

  return sc_kernel


_SC_CALL = _make_sc_call()


def kernel(x, W):
  x = x.astype(jnp.int32)
  xf = x.reshape(N, 4)
  p = xf[:, 0] * 12 + xf[:, 1]
  q = xf[:, 2] * 12 + xf[:, 3]
  t = (W[:12, None, :] + W[None, :12, :]).reshape(NUM_PAIR * D)
  out = _SC_CALL(t, p, q)
  return out.reshape(B, L, D)


# speedup vs baseline: 4.0690x; 4.0690x over previous
import functools

import jax
import jax.numpy as jnp
from jax import lax
from jax.experimental import pallas as pl
from jax.experimental.pallas import tpu as pltpu
from jax.experimental.pallas import tpu_sc as plsc

B, L, D = 4096, 200, 128
N = B * L
NUM_PAIR = 144

_INFO = plsc.get_sparse_core_info()
NC = _INFO.num_cores
NS = _INFO.num_subcores
NW = NC * NS
ROWS_PER_W = N // NW
CHUNK = 256
NCHUNK = ROWS_PER_W // CHUNK
GROUPS = CHUNK // 16
COL_BLK = 16
NCB = (D // 2) // COL_BLK
NBUF = 2


def _make_sc_call():
  mesh = plsc.VectorSubcoreMesh(core_axis_name="c", subcore_axis_name="s")

  @functools.partial(
      pl.kernel,
      out_type=jax.ShapeDtypeStruct((N * D,), jnp.float32),
      mesh=mesh,
      compiler_params=pltpu.CompilerParams(needs_layout_passes=False),
      scratch_types=[
          pltpu.VMEM((NUM_PAIR * (D // 2),), jnp.int32),
          [pltpu.VMEM((CHUNK,), jnp.int32) for _ in range(NBUF)],
          [pltpu.VMEM((CHUNK,), jnp.int32) for _ in range(NBUF)],
          [pltpu.VMEM((CHUNK * D,), jnp.float32) for _ in range(NBUF)],
          [pltpu.SemaphoreType.DMA for _ in range(NBUF)],
          [pltpu.SemaphoreType.DMA for _ in range(NBUF)],
          pltpu.SemaphoreType.DMA,
      ],
  )
  def sc_kernel(t_hbm, p_hbm, q_hbm, out_hbm, t_v, p_v, q_v, o_v,
                sem_i, sem_o, sem_t):
    wid = lax.axis_index("s") * NC + lax.axis_index("c")
    row0 = wid * ROWS_PER_W
    pltpu.async_copy(t_hbm, t_v, sem_t).wait()
    lane = lax.iota(jnp.int32, 16)
    soff0 = lane * D

    def idx_start(k, b):
      base = row0 + k * CHUNK
      pltpu.async_copy(p_hbm.at[pl.ds(base, CHUNK)], p_v[b], sem_i[b])
      pltpu.async_copy(q_hbm.at[pl.ds(base, CHUNK)], q_v[b], sem_i[b])

    def idx_wait(k, b):
      base = row0 + k * CHUNK
      pltpu.make_async_copy(p_hbm.at[pl.ds(base, CHUNK)], p_v[b], sem_i[b]).wait()
      pltpu.make_async_copy(q_hbm.at[pl.ds(base, CHUNK)], q_v[b], sem_i[b]).wait()

    def out_start(k, b):
      base = row0 + k * CHUNK
      pltpu.async_copy(o_v[b], out_hbm.at[pl.ds(base * D, CHUNK * D)], sem_o[b])

    def out_wait(k, b):
      base = row0 + k * CHUNK
      pltpu.make_async_copy(
          o_v[b], out_hbm.at[pl.ds(base * D, CHUNK * D)], sem_o[b]).wait()

    def compute(b):
      hi_mask = jnp.int32(-65536)
      h = D // 2

      @plsc.parallel_loop(0, GROUPS * NCB, 1, unroll=4)
      def group_body(i):
        g = i // NCB
        cb = i % NCB
        c0 = cb * COL_BLK
        pv = p_v[b][pl.ds(g * 16, 16)] * h
        qv = q_v[b][pl.ds(g * 16, 16)] * h
        ov = soff0 + g * (16 * D)
        for j in range(COL_BLK):
          hv = (lane + (c0 + j)) & (h - 1)
          gp = plsc.load_gather(t_v, [pv + hv])
          gq = plsc.load_gather(t_v, [qv + hv])
          r_lo = (plsc.bitcast(gp << 16, jnp.float32) +
                  plsc.bitcast(gq << 16, jnp.float32))
          r_hi = (plsc.bitcast(gp & hi_mask, jnp.float32) +
                  plsc.bitcast(gq & hi_mask, jnp.float32))
          io = ov + hv
          plsc.store_scatter(o_v[b], [io], r_lo)
          plsc.store_scatter(o_v[b], [io + h], r_hi)

    for b in range(NBUF):
      idx_start(b, b)

    def chunk_pair(kk, _):
      for b in range(NBUF):
        k = kk * NBUF + b
        idx_wait(k, b)

        @pl.when(k >= NBUF)
        def _():
          out_wait(k - NBUF, b)

        compute(b)
        out_start(k, b)

        @pl.when(k + NBUF < NCHUNK)
        def _():
          idx_start(k + NBUF, b)
      return 0

    lax.fori_loop(0, NCHUNK // NBUF, chunk_pair, 0)
    for b in range(NBUF):
      out_wait(NCHUNK - NBUF + b, b)

  return sc_kernel


_SC_CALL = _make_sc_call()


def kernel(x, W):
  x = x.astype(jnp.int32)
  xf = x.reshape(N, 4)
  p = xf[:, 0] * 12 + xf[:, 1]
  q = xf[:, 2] * 12 + xf[:, 3]
  t = (W[:12, None, :] + W[None, :12, :]).reshape(NUM_PAIR, D)
  tb = lax.bitcast_convert_type(t.astype(jnp.bfloat16), jnp.uint16).astype(
      jnp.uint32)
  tp = lax.bitcast_convert_type(
      tb[:, : D // 2] | (tb[:, D // 2:] << 16), jnp.int32).reshape(
          NUM_PAIR * (D // 2))
  out = _SC_CALL(tp, p, q)
  return out.reshape(B, L, D)


# speedup vs baseline: 4.2029x; 1.0329x over previous
import functools

import jax
import jax.numpy as jnp
from jax import lax
from jax.experimental import pallas as pl
from jax.experimental.pallas import tpu as pltpu
from jax.experimental.pallas import tpu_sc as plsc

B, L, D = 4096, 200, 128
N = B * L
NUM_PAIR = 144

_INFO = plsc.get_sparse_core_info()
NC = _INFO.num_cores
NS = _INFO.num_subcores
NW = NC * NS
ROWS_PER_W = N // NW
CHUNK = 256
NCHUNK = ROWS_PER_W // CHUNK
GROUPS = CHUNK // 16
COL_BLK = 16
NCB = (D // 2) // COL_BLK
NBUF = 2


def _make_sc_call():
  mesh = plsc.VectorSubcoreMesh(core_axis_name="c", subcore_axis_name="s")

  @functools.partial(
      pl.kernel,
      out_type=jax.ShapeDtypeStruct((N * D,), jnp.float32),
      mesh=mesh,
      compiler_params=pltpu.CompilerParams(needs_layout_passes=False),
      scratch_types=[
          pltpu.VMEM((NUM_PAIR * (D // 2),), jnp.int32),
          [pltpu.VMEM((CHUNK,), jnp.int32) for _ in range(NBUF)],
          [pltpu.VMEM((CHUNK,), jnp.int32) for _ in range(NBUF)],
          [pltpu.VMEM((CHUNK * D,), jnp.float32) for _ in range(NBUF)],
          [pltpu.SemaphoreType.DMA for _ in range(NBUF)],
          [pltpu.SemaphoreType.DMA for _ in range(NBUF)],
          pltpu.SemaphoreType.DMA,
      ],
  )
  def sc_kernel(t_hbm, p_hbm, q_hbm, out_hbm, t_v, p_v, q_v, o_v,
                sem_i, sem_o, sem_t):
    wid = lax.axis_index("s") * NC + lax.axis_index("c")
    row0 = wid * ROWS_PER_W
    pltpu.async_copy(t_hbm, t_v, sem_t).wait()
    lane = lax.iota(jnp.int32, 16)
    soff0 = lane * D

    def idx_start(k, b):
      base = row0 + k * CHUNK
      pltpu.async_copy(p_hbm.at[pl.ds(base, CHUNK)], p_v[b], sem_i[b])
      pltpu.async_copy(q_hbm.at[pl.ds(base, CHUNK)], q_v[b], sem_i[b])

    def idx_wait(k, b):
      base = row0 + k * CHUNK
      pltpu.make_async_copy(p_hbm.at[pl.ds(base, CHUNK)], p_v[b], sem_i[b]).wait()
      pltpu.make_async_copy(q_hbm.at[pl.ds(base, CHUNK)], q_v[b], sem_i[b]).wait()

    def out_start(k, b):
      base = row0 + k * CHUNK
      pltpu.async_copy(o_v[b], out_hbm.at[pl.ds(base * D, CHUNK * D)], sem_o[b])

    def out_wait(k, b):
      base = row0 + k * CHUNK
      pltpu.make_async_copy(
          o_v[b], out_hbm.at[pl.ds(base * D, CHUNK * D)], sem_o[b]).wait()

    def compute(b):
      hi_mask = jnp.int32(-65536)
      h = D // 2

      @plsc.parallel_loop(0, GROUPS * NCB, 1, unroll=4)
      def group_body(i):
        g = i // NCB
        cb = i % NCB
        c0 = cb * COL_BLK
        pv = p_v[b][pl.ds(g * 16, 16)] * h
        qv = q_v[b][pl.ds(g * 16, 16)] * h
        ov = soff0 + g * (16 * D)
        for bb in range(COL_BLK // 4):
          cve = ((lane + (c0 // 4 + bb)) & 15) * 2
          ip0 = pv + cve
          iq0 = qv + cve
          io0 = ov + cve
          for u in range(2):
            for j2 in range(2):
              off = u * 32 + j2
              gp = plsc.load_gather(t_v, [ip0 + off])
              gq = plsc.load_gather(t_v, [iq0 + off])
              r_lo = (plsc.bitcast(gp << 16, jnp.float32) +
                      plsc.bitcast(gq << 16, jnp.float32))
              r_hi = (plsc.bitcast(gp & hi_mask, jnp.float32) +
                      plsc.bitcast(gq & hi_mask, jnp.float32))
              io = io0 + off
              plsc.store_scatter(o_v[b], [io], r_lo)
              plsc.store_scatter(o_v[b], [io + h], r_hi)

    for b in range(NBUF):
      idx_start(b, b)

    def chunk_pair(kk, _):
      for b in range(NBUF):
        k = kk * NBUF + b
        idx_wait(k, b)

        @pl.when(k >= NBUF)
        def _():
          out_wait(k - NBUF, b)

        compute(b)
        out_start(k, b)

        @pl.when(k + NBUF < NCHUNK)
        def _():
          idx_start(k + NBUF, b)
      return 0

    lax.fori_loop(0, NCHUNK // NBUF, chunk_pair, 0)
    for b in range(NBUF):
      out_wait(NCHUNK - NBUF + b, b)

  return sc_kernel


_SC_CALL = _make_sc_call()


def kernel(x, W):
  x = x.astype(jnp.int32)
  xf = x.reshape(N, 4)
  p = xf[:, 0] * 12 + xf[:, 1]
  q = xf[:, 2] * 12 + xf[:, 3]
  t = (W[:12, None, :] + W[None, :12, :]).reshape(NUM_PAIR, D)
  tb = lax.bitcast_convert_type(t.astype(jnp.bfloat16), jnp.uint16).astype(
      jnp.uint32)
  tp = lax.bitcast_convert_type(
      tb[:, : D // 2] | (tb[:, D // 2:] << 16), jnp.int32).reshape(
          NUM_PAIR * (D // 2))
  out = _SC_CALL(tp, p, q)
  return out.reshape(B, L, D)
